# baseline (speedup 1.0000x reference)
import functools

import jax
import jax.numpy as jnp
from jax import lax
from jax.experimental import pallas as pl
from jax.experimental.pallas import tpu as pltpu
from jax.experimental.pallas import tpu_sc as plsc

B, T, L, H, D = 64, 64, 577, 12, 768
LNC = L - 1
K = LNC // 2
KP1 = K + 1
NIDX = 384
CHUNK = 128
NW = 32


def _score_kernel(ca_ref, tm_ref, im_ref, ts_ref, idx_ref, mask_ref):
    b = pl.program_id(0)
    X = ca_ref[0]

    acc = X[:, 0:8, :]
    for g in range(1, 8):
        acc = acc + X[:, 8 * g:8 * g + 8, :]
    t1 = acc[:, 4:8, :] + acc[:, 0:4, :]
    t2 = t1[:, 2:4, :] + t1[:, 0:2, :]
    S = t2[:, 1, :] + t2[:, 0, :]

    t_len = jnp.sum(tm_ref[...])
    Sd = S / t_len

    ts_ref[0] = Sd.T

    A = Sd[0:4, :] + Sd[8:12, :]
    a = Sd[4:8, :] + A
    bb = a[2:4, :] + a[0:2, :]
    c = bb[1:2, :] + bb[0:1, :]
    scores = c / jnp.float32(12.0)

    s_nc = scores[:, 1:]

    row = jnp.broadcast_to(s_nc, (LNC, LNC))
    ii = lax.broadcasted_iota(jnp.int32, (LNC, LNC), 0)
    jj = lax.broadcasted_iota(jnp.int32, (LNC, LNC), 1)
    colm = jnp.where(ii == jj, row, 0.0)
    ones = jnp.ones((LNC, 1), jnp.float32)
    col = lax.dot_general(colm, ones, (((1,), (0,)), ((), ())),
                          precision=lax.Precision.HIGHEST)
    colb = jnp.broadcast_to(col, (LNC, LNC))
    G = (row > colb) | ((row == colb) & (jj < ii))
    rank = lax.dot_general(G.astype(jnp.float32), ones,
                           (((1,), (0,)), ((), ())),
                           precision=lax.Precision.HIGHEST)

    rr = lax.broadcasted_iota(jnp.int32, (LNC, NIDX), 1).astype(jnp.float32)
    P2 = (jnp.broadcast_to(rank + 1.0, (LNC, NIDX)) == rr).astype(jnp.float32)

    idx_row = lax.broadcasted_iota(jnp.int32, (1, LNC), 1).astype(jnp.float32)
    mask_row = im_ref[0][:, 1:]
    lhs = jnp.concatenate([idx_row, mask_row], axis=0)
    packed = lax.dot_general(lhs, P2, (((1,), (0,)), ((), ())),
                             precision=lax.Precision.HIGHEST)
    top_shift = packed[0:1, :]
    gmask = packed[1:2, :]

    lane = lax.broadcasted_iota(jnp.int32, (1, NIDX), 1)
    base = (b * L).astype(jnp.float32)
    gidx = jnp.where(lane == 0, base, base + 1.0 + top_shift)
    gidx = jnp.where(lane < KP1, gidx, base).astype(jnp.int32)
    idx_ref[0] = gidx

    cls_mask = im_ref[0, 0, 0]
    mrow = jnp.where(lane == 0, cls_mask, gmask)
    mask_ref[0] = mrow[:, :KP1]


def _tc_scores(cross_attn, text_mask, image_mask):
    return pl.pallas_call(
        _score_kernel,
        grid=(B,),
        in_specs=[
            pl.BlockSpec((1, H, T, L), lambda b: (b, 0, 0, 0)),
            pl.BlockSpec((1, 1, T), lambda b: (b, 0, 0)),
            pl.BlockSpec((1, 1, L), lambda b: (b, 0, 0)),
        ],
        out_specs=[
            pl.BlockSpec((1, L, H), lambda b: (b, 0, 0)),
            pl.BlockSpec((1, 1, NIDX), lambda b: (b, 0, 0)),
            pl.BlockSpec((1, 1, KP1), lambda b: (b, 0, 0)),
        ],
        out_shape=[
            jax.ShapeDtypeStruct((B, L, H), jnp.float32),
            jax.ShapeDtypeStruct((B, 1, NIDX), jnp.int32),
            jax.ShapeDtypeStruct((B, 1, KP1), jnp.float32),
        ],
    )(cross_attn, text_mask.reshape(B, 1, T), image_mask.reshape(B, 1, L))


def _make_sc_gather():
    mesh = plsc.VectorSubcoreMesh(core_axis_name="c", subcore_axis_name="s")

    @functools.partial(
        pl.kernel,
        mesh=mesh,
        out_type=jax.ShapeDtypeStruct((B, KP1, D), jnp.float32),
        scratch_types=[
            pltpu.VMEM((CHUNK,), jnp.int32),
            pltpu.VMEM((CHUNK, D), jnp.float32),
            pltpu.SemaphoreType.DMA,
        ],
        compiler_params=pltpu.CompilerParams(use_tc_tiling_on_sc=False),
    )
    def sc_gather(table_hbm, idx_hbm, out_hbm, idx_v, rows_v, sem):
        wid = lax.axis_index("s") * 2 + lax.axis_index("c")
        for bo in range(2):
            b = wid * 2 + bo
            for ch in range(3):
                pltpu.sync_copy(idx_hbm.at[pl.ds((b * 3 + ch) * CHUNK, CHUNK)],
                                idx_v)
                pltpu.async_copy(table_hbm.at[idx_v], rows_v, sem).wait()
                if ch < 2:
                    pltpu.sync_copy(rows_v, out_hbm.at[b, pl.ds(ch * CHUNK, CHUNK)])
                else:
                    n = KP1 - 2 * CHUNK
                    pltpu.sync_copy(rows_v.at[pl.ds(0, n)],
                                    out_hbm.at[b, pl.ds(ch * CHUNK, n)])

    return sc_gather


_SC_GATHER_CACHE = []


def kernel(layer_idx, text_states, text_mask, image_states, image_mask,
           cross_attn, previous_keep_mask):
    token_scores, gidx, new_mask = _tc_scores(cross_attn, text_mask, image_mask)
    table = image_states.reshape(B * L, D)
    idx_flat = gidx.reshape(B * NIDX)
    if not _SC_GATHER_CACHE:
        _SC_GATHER_CACHE.append(_make_sc_gather())
    new_img_states = _SC_GATHER_CACHE[0](table, idx_flat)
    new_img_mask = new_mask.reshape(B, KP1)
    return (new_img_states, new_img_mask, previous_keep_mask, token_scores)


# speedup vs baseline: 1.4214x; 1.4214x over previous
import functools

import jax
import jax.numpy as jnp
from jax import lax
from jax.experimental import pallas as pl
from jax.experimental.pallas import tpu as pltpu
from jax.experimental.pallas import tpu_sc as plsc

B, T, L, H, D = 64, 64, 577, 12, 768
LNC = L - 1
K = LNC // 2
KP1 = K + 1
NIDX = 384
CHUNK = 128
NW = 32


def _score_kernel(ca_ref, tm_ref, im_ref, ts_ref, idx_ref, mask_ref):
    b = pl.program_id(0)
    X = ca_ref[0]

    acc = X[:, 0:8, :]
    for g in range(1, 8):
        acc = acc + X[:, 8 * g:8 * g + 8, :]
    t1 = acc[:, 4:8, :] + acc[:, 0:4, :]
    t2 = t1[:, 2:4, :] + t1[:, 0:2, :]
    S = t2[:, 1, :] + t2[:, 0, :]

    t_len = jnp.sum(tm_ref[...])
    Sd = S / t_len

    SdT = Sd.T
    ts_ref[0] = SdT

    A = Sd[0:4, :] + Sd[8:12, :]
    a = Sd[4:8, :] + A
    bb = a[2:4, :] + a[0:2, :]
    c = bb[1:2, :] + bb[0:1, :]
    scores = c / jnp.float32(12.0)
    s_nc = scores[:, 1:]

    Ac = SdT[:, 0:4] + SdT[:, 8:12]
    ac = SdT[:, 4:8] + Ac
    bc = ac[:, 2:4] + ac[:, 0:2]
    cc = bc[:, 1:2] + bc[:, 0:1]
    col = (cc / jnp.float32(12.0))[1:, :]

    row = jnp.broadcast_to(s_nc, (LNC, LNC))
    ii = lax.broadcasted_iota(jnp.int32, (LNC, LNC), 0)
    jj = lax.broadcasted_iota(jnp.int32, (LNC, LNC), 1)
    ones = jnp.ones((LNC, 1), jnp.float32)
    colb = jnp.broadcast_to(col, (LNC, LNC))
    G = (row > colb) | ((row == colb) & (jj < ii))
    rank = lax.dot_general(G.astype(jnp.float32), ones,
                           (((1,), (0,)), ((), ())))

    rr = lax.broadcasted_iota(jnp.int32, (LNC, NIDX), 1).astype(jnp.float32)
    P2 = (jnp.broadcast_to(rank + 1.0, (LNC, NIDX)) == rr).astype(jnp.float32)

    idx_i = lax.broadcasted_iota(jnp.int32, (1, LNC), 1)
    idx_hi = (idx_i // 64).astype(jnp.float32)
    idx_lo = (idx_i % 64).astype(jnp.float32)
    mask_row = im_ref[0][:, 1:]
    lhs = jnp.concatenate([idx_hi, idx_lo, mask_row], axis=0)
    packed = lax.dot_general(lhs, P2, (((1,), (0,)), ((), ())))
    top_shift = packed[0:1, :] * 64.0 + packed[1:2, :]
    gmask = packed[2:3, :]

    lane = lax.broadcasted_iota(jnp.int32, (1, NIDX), 1)
    base = (b * L).astype(jnp.float32)
    gidx = jnp.where(lane == 0, base, base + 1.0 + top_shift)
    gidx = jnp.where(lane < KP1, gidx, base).astype(jnp.int32)
    idx_ref[0] = gidx

    cls_mask = im_ref[0, 0, 0]
    mrow = jnp.where(lane == 0, cls_mask, gmask)
    mask_ref[0] = mrow[:, :KP1]


def _tc_scores(cross_attn, text_mask, image_mask):
    return pl.pallas_call(
        _score_kernel,
        grid=(B,),
        in_specs=[
            pl.BlockSpec((1, H, T, L), lambda b: (b, 0, 0, 0)),
            pl.BlockSpec((1, 1, T), lambda b: (b, 0, 0)),
            pl.BlockSpec((1, 1, L), lambda b: (b, 0, 0)),
        ],
        out_specs=[
            pl.BlockSpec((1, L, H), lambda b: (b, 0, 0)),
            pl.BlockSpec((1, 1, NIDX), lambda b: (b, 0, 0)),
            pl.BlockSpec((1, 1, KP1), lambda b: (b, 0, 0)),
        ],
        out_shape=[
            jax.ShapeDtypeStruct((B, L, H), jnp.float32),
            jax.ShapeDtypeStruct((B, 1, NIDX), jnp.int32),
            jax.ShapeDtypeStruct((B, 1, KP1), jnp.float32),
        ],
    )(cross_attn, text_mask.reshape(B, 1, T), image_mask.reshape(B, 1, L))


NROWS = B * KP1
NCHUNK = (NROWS + CHUNK - 1) // CHUNK
NTAIL = NROWS - (NCHUNK - 1) * CHUNK


def _make_sc_gather():
    mesh = plsc.VectorSubcoreMesh(core_axis_name="c", subcore_axis_name="s")

    @functools.partial(
        pl.kernel,
        mesh=mesh,
        out_type=jax.ShapeDtypeStruct((NROWS, D), jnp.float32),
        scratch_types=[
            pltpu.VMEM((CHUNK,), jnp.int32),
            pltpu.VMEM((CHUNK, D), jnp.float32),
            pltpu.SemaphoreType.DMA,
        ],
    )
    def sc_gather(table_hbm, idx_hbm, out_hbm, idx_v, rows_v, sem):
        w = lax.axis_index("s") * 2 + lax.axis_index("c")

        def do_chunk(ch, nwrite):
            pltpu.sync_copy(idx_hbm.at[pl.ds(ch * CHUNK, CHUNK)], idx_v)
            pltpu.async_copy(table_hbm.at[idx_v], rows_v, sem).wait()
            if nwrite == CHUNK:
                pltpu.sync_copy(rows_v, out_hbm.at[pl.ds(ch * CHUNK, CHUNK)])
            else:
                pltpu.sync_copy(rows_v.at[pl.ds(0, nwrite)],
                                out_hbm.at[pl.ds(ch * CHUNK, nwrite)])

        for t in range(4):
            ch = jnp.where(w < 16, 5 * w + t, 80 + 4 * (w - 16) + t)
            do_chunk(ch, CHUNK)

        @pl.when(w < 16)
        def _():
            do_chunk(5 * w + 4, CHUNK)

        @pl.when(w == 31)
        def _():
            do_chunk(NCHUNK - 1, NTAIL)

    return sc_gather


_SC_GATHER_CACHE = []


def kernel(layer_idx, text_states, text_mask, image_states, image_mask,
           cross_attn, previous_keep_mask):
    token_scores, gidx, new_mask = _tc_scores(cross_attn, text_mask, image_mask)
    table = image_states.reshape(B * L, D)
    idx_flat = gidx.reshape(B, NIDX)[:, :KP1].reshape(B * KP1)
    idx_pad = jnp.concatenate(
        [idx_flat, jnp.zeros((NCHUNK * CHUNK - NROWS,), jnp.int32)])
    if not _SC_GATHER_CACHE:
        _SC_GATHER_CACHE.append(_make_sc_gather())
    new_img_states = _SC_GATHER_CACHE[0](table, idx_pad).reshape(B, KP1, D)
    new_img_mask = new_mask.reshape(B, KP1)
    return (new_img_states, new_img_mask, previous_keep_mask, token_scores)


# speedup vs baseline: 1.8363x; 1.2919x over previous
import functools

import jax
import jax.numpy as jnp
from jax import lax
from jax.experimental import pallas as pl
from jax.experimental.pallas import tpu as pltpu
from jax.experimental.pallas import tpu_sc as plsc

B, T, L, H, D = 64, 64, 577, 12, 768
LNC = L - 1
K = LNC // 2
KP1 = K + 1
NIDX = 384
CHUNK = 128
NW = 32


def _score_kernel(ca_ref, tm_ref, im_ref, ts_ref, idx_ref, mask_ref):
    b = pl.program_id(0)
    X = ca_ref[0]

    acc = X[:, 0:8, :]
    for g in range(1, 8):
        acc = acc + X[:, 8 * g:8 * g + 8, :]
    t1 = acc[:, 4:8, :] + acc[:, 0:4, :]
    t2 = t1[:, 2:4, :] + t1[:, 0:2, :]
    S = t2[:, 1, :] + t2[:, 0, :]

    t_len = jnp.sum(tm_ref[...])
    Sd = S / t_len

    SdT = Sd.T
    ts_ref[0] = SdT

    A = Sd[0:4, :] + Sd[8:12, :]
    a = Sd[4:8, :] + A
    bb = a[2:4, :] + a[0:2, :]
    c = bb[1:2, :] + bb[0:1, :]
    scores = c / jnp.float32(12.0)
    s_nc = scores[:, 1:]

    Ac = SdT[:, 0:4] + SdT[:, 8:12]
    ac = SdT[:, 4:8] + Ac
    bc = ac[:, 2:4] + ac[:, 0:2]
    cc = bc[:, 1:2] + bc[:, 0:1]
    col = (cc / jnp.float32(12.0))[1:, :]

    row = jnp.broadcast_to(s_nc, (LNC, LNC))
    ii = lax.broadcasted_iota(jnp.int32, (LNC, LNC), 0)
    jj = lax.broadcasted_iota(jnp.int32, (LNC, LNC), 1)
    ones = jnp.ones((LNC, 1), jnp.float32)
    colb = jnp.broadcast_to(col, (LNC, LNC))
    G = (row > colb) | ((row == colb) & (jj < ii))
    rank = lax.dot_general(G.astype(jnp.float32), ones,
                           (((1,), (0,)), ((), ())))

    rr_l = lax.broadcasted_iota(jnp.int32, (LNC, NIDX), 1)
    rr = jnp.where(rr_l < KP1, rr_l, rr_l - KP1).astype(jnp.float32)
    P2 = (jnp.broadcast_to(rank + 1.0, (LNC, NIDX)) == rr).astype(jnp.float32)

    idx_i = lax.broadcasted_iota(jnp.int32, (1, LNC), 1)
    idx_hi = (idx_i // 64).astype(jnp.float32)
    idx_lo = (idx_i % 64).astype(jnp.float32)
    mask_row = im_ref[0][:, 1:]
    lhs = jnp.concatenate([idx_hi, idx_lo, mask_row], axis=0)
    packed = lax.dot_general(lhs, P2, (((1,), (0,)), ((), ())))
    top_shift = packed[0:1, :] * 64.0 + packed[1:2, :]
    gmask = packed[2:3, :]

    lane = lax.broadcasted_iota(jnp.int32, (1, NIDX), 1)
    lane_t = jnp.where(lane < KP1, lane, lane - KP1)
    gidx = jnp.where(lane_t == 0, 0.0, 1.0 + top_shift).astype(jnp.int32)
    idx_ref[0] = gidx

    cls_mask = im_ref[0, 0, 0]
    mrow = jnp.where(lane == 0, cls_mask, gmask)
    mask_ref[0] = mrow[:, :KP1]


def _tc_scores(cross_attn, text_mask, image_mask):
    return pl.pallas_call(
        _score_kernel,
        grid=(B,),
        in_specs=[
            pl.BlockSpec((1, H, T, L), lambda b: (b, 0, 0, 0)),
            pl.BlockSpec((1, 1, T), lambda b: (b, 0, 0)),
            pl.BlockSpec((1, 1, L), lambda b: (b, 0, 0)),
        ],
        out_specs=[
            pl.BlockSpec((1, L, H), lambda b: (b, 0, 0)),
            pl.BlockSpec((1, 1, NIDX), lambda b: (b, 0, 0)),
            pl.BlockSpec((1, 1, KP1), lambda b: (b, 0, 0)),
        ],
        out_shape=[
            jax.ShapeDtypeStruct((B, L, H), jnp.float32),
            jax.ShapeDtypeStruct((B, 1, NIDX), jnp.int32),
            jax.ShapeDtypeStruct((B, 1, KP1), jnp.float32),
        ],
    )(cross_attn, text_mask.reshape(B, 1, T), image_mask.reshape(B, 1, L))


NTAIL = KP1 - 2 * CHUNK


def _make_sc_gather():
    mesh = plsc.VectorSubcoreMesh(core_axis_name="c", subcore_axis_name="s")

    @functools.partial(
        pl.kernel,
        mesh=mesh,
        out_type=jax.ShapeDtypeStruct((B, KP1, D), jnp.float32),
        scratch_types=[
            pltpu.VMEM((CHUNK,), jnp.int32),
            pltpu.VMEM((CHUNK, D), jnp.float32),
            pltpu.VMEM((CHUNK,), jnp.int32),
            pltpu.SemaphoreType.DMA,
        ],
    )
    def sc_gather(img_hbm, idx_hbm, out_hbm, idx_v, rows_v, wtail, sem):
        w = lax.axis_index("s") * 2 + lax.axis_index("c")

        i16 = lax.broadcasted_iota(jnp.int32, (16,), 0)
        for k in range(CHUNK // 16):
            v = i16 + 16 * k
            wtail[pl.ds(16 * k, 16)] = jnp.where(
                v < NTAIL, v + 2 * CHUNK, v - NTAIL)

        for bo in range(2):
            b = w * 2 + bo
            imgb = img_hbm.at[b]
            outb = out_hbm.at[b]
            for ch in range(3):
                pltpu.sync_copy(
                    idx_hbm.at[pl.ds(b * NIDX + ch * CHUNK, CHUNK)], idx_v)
                pltpu.async_copy(imgb.at[idx_v], rows_v, sem).wait()
                if ch < 2:
                    pltpu.sync_copy(rows_v, outb.at[pl.ds(ch * CHUNK, CHUNK)])
                else:
                    pltpu.async_copy(rows_v, outb.at[wtail], sem).wait()

    return sc_gather


_SC_GATHER_CACHE = []


def kernel(layer_idx, text_states, text_mask, image_states, image_mask,
           cross_attn, previous_keep_mask):
    token_scores, gidx, new_mask = _tc_scores(cross_attn, text_mask, image_mask)
    idx_flat = gidx.reshape(B * NIDX)
    if not _SC_GATHER_CACHE:
        _SC_GATHER_CACHE.append(_make_sc_gather())
    new_img_states = _SC_GATHER_CACHE[0](image_states, idx_flat)
    new_img_mask = new_mask.reshape(B, KP1)
    return (new_img_states, new_img_mask, previous_keep_mask, token_scores)


# speedup vs baseline: 1.9407x; 1.0569x over previous
import functools

import jax
import jax.numpy as jnp
from jax import lax
from jax.experimental import pallas as pl
from jax.experimental.pallas import tpu as pltpu
from jax.experimental.pallas import tpu_sc as plsc

B, T, L, H, D = 64, 64, 577, 12, 768
LNC = L - 1
K = LNC // 2
KP1 = K + 1
NIDX = 384
CHUNK = 128
NW = 32


def _score_kernel(ca_ref, tm_ref, im_ref, ts_ref, idx_ref, mask_ref):
    b = pl.program_id(0)
    X = ca_ref[0]

    acc = X[:, 0:8, :]
    for g in range(1, 8):
        acc = acc + X[:, 8 * g:8 * g + 8, :]
    t1 = acc[:, 4:8, :] + acc[:, 0:4, :]
    t2 = t1[:, 2:4, :] + t1[:, 0:2, :]
    S = t2[:, 1, :] + t2[:, 0, :]

    t_len = jnp.sum(tm_ref[...])
    Sd = S / t_len

    SdT = Sd.T
    ts_ref[0] = SdT

    A = Sd[0:4, :] + Sd[8:12, :]
    a = Sd[4:8, :] + A
    bb = a[2:4, :] + a[0:2, :]
    c = bb[1:2, :] + bb[0:1, :]
    scores = c / jnp.float32(12.0)
    s_nc = scores[:, 1:]

    Ac = SdT[:, 0:4] + SdT[:, 8:12]
    ac = SdT[:, 4:8] + Ac
    bc = ac[:, 2:4] + ac[:, 0:2]
    cc = bc[:, 1:2] + bc[:, 0:1]
    col = (cc / jnp.float32(12.0))[1:, :]

    krow = jnp.broadcast_to(
        lax.bitcast_convert_type(s_nc, jnp.int32), (LNC, LNC))
    kcol = jnp.broadcast_to(
        lax.bitcast_convert_type(col, jnp.int32), (LNC, LNC))
    ii = lax.broadcasted_iota(jnp.int32, (LNC, LNC), 0)
    jj = lax.broadcasted_iota(jnp.int32, (LNC, LNC), 1)
    tri = jnp.where(jj < ii, 1, 0)
    G = (krow - kcol + tri) > 0
    ones = jnp.ones((LNC, 1), jnp.bfloat16)
    rank = lax.dot_general(G.astype(jnp.bfloat16), ones,
                           (((1,), (0,)), ((), ())),
                           preferred_element_type=jnp.float32)

    rr_l = lax.broadcasted_iota(jnp.int32, (LNC, NIDX), 1)
    rr = jnp.where(rr_l < KP1, rr_l, rr_l - KP1).astype(jnp.float32)
    P2 = (jnp.broadcast_to(rank + 1.0, (LNC, NIDX)) == rr).astype(jnp.bfloat16)

    idx_i = lax.broadcasted_iota(jnp.int32, (1, LNC), 1)
    idx_hi = (idx_i // 64).astype(jnp.bfloat16)
    idx_lo = (idx_i % 64).astype(jnp.bfloat16)
    mask_row = im_ref[0][:, 1:].astype(jnp.bfloat16)
    lhs = jnp.concatenate([idx_hi, idx_lo, mask_row], axis=0)
    packed = lax.dot_general(lhs, P2, (((1,), (0,)), ((), ())),
                             preferred_element_type=jnp.float32)
    top_shift = packed[0:1, :] * 64.0 + packed[1:2, :]
    gmask = packed[2:3, :]

    lane = lax.broadcasted_iota(jnp.int32, (1, NIDX), 1)
    lane_t = jnp.where(lane < KP1, lane, lane - KP1)
    gidx = jnp.where(lane_t == 0, 0.0, 1.0 + top_shift).astype(jnp.int32)
    idx_ref[0] = gidx

    cls_mask = im_ref[0, 0, 0]
    mrow = jnp.where(lane == 0, cls_mask, gmask)
    mask_ref[0] = mrow[:, :KP1]


def _tc_scores(cross_attn, text_mask, image_mask):
    return pl.pallas_call(
        _score_kernel,
        grid=(B,),
        in_specs=[
            pl.BlockSpec((1, H, T, L), lambda b: (b, 0, 0, 0)),
            pl.BlockSpec((1, 1, T), lambda b: (b, 0, 0)),
            pl.BlockSpec((1, 1, L), lambda b: (b, 0, 0)),
        ],
        out_specs=[
            pl.BlockSpec((1, L, H), lambda b: (b, 0, 0)),
            pl.BlockSpec((1, 1, NIDX), lambda b: (b, 0, 0)),
            pl.BlockSpec((1, 1, KP1), lambda b: (b, 0, 0)),
        ],
        out_shape=[
            jax.ShapeDtypeStruct((B, L, H), jnp.float32),
            jax.ShapeDtypeStruct((B, 1, NIDX), jnp.int32),
            jax.ShapeDtypeStruct((B, 1, KP1), jnp.float32),
        ],
    )(cross_attn, text_mask.reshape(B, 1, T), image_mask.reshape(B, 1, L))


NTAIL = KP1 - 2 * CHUNK


def _make_sc_gather():
    mesh = plsc.VectorSubcoreMesh(core_axis_name="c", subcore_axis_name="s")

    @functools.partial(
        pl.kernel,
        mesh=mesh,
        out_type=jax.ShapeDtypeStruct((B, KP1, D), jnp.float32),
        scratch_types=[
            pltpu.VMEM((CHUNK,), jnp.int32),
            pltpu.VMEM((CHUNK, D), jnp.float32),
            pltpu.VMEM((CHUNK,), jnp.int32),
            pltpu.SemaphoreType.DMA,
        ],
    )
    def sc_gather(img_hbm, idx_hbm, out_hbm, idx_v, rows_v, wtail, sem):
        w = lax.axis_index("s") * 2 + lax.axis_index("c")

        i16 = lax.broadcasted_iota(jnp.int32, (16,), 0)
        for k in range(CHUNK // 16):
            v = i16 + 16 * k
            wtail[pl.ds(16 * k, 16)] = jnp.where(
                v < NTAIL, v + 2 * CHUNK, v - NTAIL)

        for bo in range(2):
            b = w * 2 + bo
            imgb = img_hbm.at[b]
            outb = out_hbm.at[b]
            for ch in range(3):
                pltpu.sync_copy(
                    idx_hbm.at[pl.ds(b * NIDX + ch * CHUNK, CHUNK)], idx_v)
                pltpu.async_copy(imgb.at[idx_v], rows_v, sem).wait()
                if ch < 2:
                    pltpu.sync_copy(rows_v, outb.at[pl.ds(ch * CHUNK, CHUNK)])
                else:
                    pltpu.async_copy(rows_v, outb.at[wtail], sem).wait()

    return sc_gather


_SC_GATHER_CACHE = []


def kernel(layer_idx, text_states, text_mask, image_states, image_mask,
           cross_attn, previous_keep_mask):
    token_scores, gidx, new_mask = _tc_scores(cross_attn, text_mask, image_mask)
    idx_flat = gidx.reshape(B * NIDX)
    if not _SC_GATHER_CACHE:
        _SC_GATHER_CACHE.append(_make_sc_gather())
    new_img_states = _SC_GATHER_CACHE[0](image_states, idx_flat)
    new_img_mask = new_mask.reshape(B, KP1)
    return (new_img_states, new_img_mask, previous_keep_mask, token_scores)


# speedup vs baseline: 1.9660x; 1.0130x over previous
import functools

import jax
import jax.numpy as jnp
from jax import lax
from jax.experimental import pallas as pl
from jax.experimental.pallas import tpu as pltpu
from jax.experimental.pallas import tpu_sc as plsc

B, T, L, H, D = 64, 64, 577, 12, 768
LNC = L - 1
K = LNC // 2
KP1 = K + 1
NIDX = 512
CHUNK = 128
NW = 32


def _score_kernel(ca_ref, tm_ref, im_ref, ts_ref, idx_ref, mask_ref):
    b = pl.program_id(0)
    X = ca_ref[0]

    acc = X[:, 0:8, :]
    for g in range(1, 8):
        acc = acc + X[:, 8 * g:8 * g + 8, :]
    t1 = acc[:, 4:8, :] + acc[:, 0:4, :]
    t2 = t1[:, 2:4, :] + t1[:, 0:2, :]
    S = t2[:, 1, :] + t2[:, 0, :]

    t_len = jnp.sum(tm_ref[...])
    Sd = S / t_len

    ts_ref[0] = Sd.T

    A = Sd[0:4, :] + Sd[8:12, :]
    a = Sd[4:8, :] + A
    bb = a[2:4, :] + a[0:2, :]
    c = bb[1:2, :] + bb[0:1, :]
    scores = c / jnp.float32(12.0)
    s_nc = scores[:, 1:]

    col = s_nc.T

    krow = jnp.broadcast_to(
        lax.bitcast_convert_type(s_nc, jnp.int32), (LNC, LNC))
    kcol = jnp.broadcast_to(
        lax.bitcast_convert_type(col, jnp.int32), (LNC, LNC))
    ii = lax.broadcasted_iota(jnp.int32, (LNC, LNC), 0)
    jj = lax.broadcasted_iota(jnp.int32, (LNC, LNC), 1)
    tri = jnp.where(jj < ii, 1, 0)
    G = (krow - kcol + tri) > 0
    ones = jnp.ones((LNC, 1), jnp.bfloat16)
    rank = lax.dot_general(G.astype(jnp.bfloat16), ones,
                           (((1,), (0,)), ((), ())),
                           preferred_element_type=jnp.float32)

    rr_l = lax.broadcasted_iota(jnp.int32, (LNC, NIDX), 1)
    rr = jnp.where(rr_l < KP1, rr_l, rr_l - KP1).astype(jnp.float32)
    P2 = (jnp.broadcast_to(rank + 1.0, (LNC, NIDX)) == rr).astype(jnp.bfloat16)

    idx_i = lax.broadcasted_iota(jnp.int32, (1, LNC), 1)
    idx_hi = (idx_i // 64).astype(jnp.bfloat16)
    idx_lo = (idx_i % 64).astype(jnp.bfloat16)
    mask_row = im_ref[0][:, 1:].astype(jnp.bfloat16)
    lhs = jnp.concatenate([idx_hi, idx_lo, mask_row], axis=0)
    packed = lax.dot_general(lhs, P2, (((1,), (0,)), ((), ())),
                             preferred_element_type=jnp.float32)
    top_shift = packed[0:1, :] * 64.0 + packed[1:2, :]
    gmask = packed[2:3, :]

    lane = lax.broadcasted_iota(jnp.int32, (1, NIDX), 1)
    lane_t = jnp.where(lane < KP1, lane, lane - KP1)
    gidx = jnp.where(lane_t == 0, 0.0, 1.0 + top_shift).astype(jnp.int32)
    idx_ref[...] = gidx[0]

    cls_mask = im_ref[0, 0, 0]
    mrow = jnp.where(lane == 0, cls_mask, gmask)
    mask_ref[0] = mrow[:, :KP1]


def _tc_scores(cross_attn, text_mask, image_mask):
    return pl.pallas_call(
        _score_kernel,
        grid=(B,),
        in_specs=[
            pl.BlockSpec((1, H, T, L), lambda b: (b, 0, 0, 0)),
            pl.BlockSpec((1, 1, T), lambda b: (b, 0, 0)),
            pl.BlockSpec((1, 1, L), lambda b: (b, 0, 0)),
        ],
        out_specs=[
            pl.BlockSpec((1, L, H), lambda b: (b, 0, 0)),
            pl.BlockSpec((NIDX,), lambda b: (b,)),
            pl.BlockSpec((1, 1, KP1), lambda b: (b, 0, 0)),
        ],
        out_shape=[
            jax.ShapeDtypeStruct((B, L, H), jnp.float32),
            jax.ShapeDtypeStruct((B * NIDX,), jnp.int32),
            jax.ShapeDtypeStruct((B, 1, KP1), jnp.float32),
        ],
    )(cross_attn, text_mask.reshape(B, 1, T), image_mask.reshape(B, 1, L))


NTAIL = KP1 - 2 * CHUNK


def _make_sc_gather():
    mesh = plsc.VectorSubcoreMesh(core_axis_name="c", subcore_axis_name="s")

    @functools.partial(
        pl.kernel,
        mesh=mesh,
        out_type=jax.ShapeDtypeStruct((B, KP1, D), jnp.float32),
        scratch_types=[
            pltpu.VMEM((CHUNK,), jnp.int32),
            pltpu.VMEM((CHUNK, D), jnp.float32),
            pltpu.VMEM((CHUNK,), jnp.int32),
            pltpu.SemaphoreType.DMA,
        ],
    )
    def sc_gather(img_hbm, idx_hbm, out_hbm, idx_v, rows_v, wtail, sem):
        w = lax.axis_index("s") * 2 + lax.axis_index("c")

        i16 = lax.broadcasted_iota(jnp.int32, (16,), 0)
        for k in range(CHUNK // 16):
            v = i16 + 16 * k
            wtail[pl.ds(16 * k, 16)] = jnp.where(
                v < NTAIL, v + 2 * CHUNK, v - NTAIL)

        for bo in range(2):
            b = w * 2 + bo
            imgb = img_hbm.at[b]
            outb = out_hbm.at[b]
            for ch in range(3):
                pltpu.sync_copy(
                    idx_hbm.at[pl.ds(b * NIDX + ch * CHUNK, CHUNK)], idx_v)
                pltpu.async_copy(imgb.at[idx_v], rows_v, sem).wait()
                if ch < 2:
                    pltpu.sync_copy(rows_v, outb.at[pl.ds(ch * CHUNK, CHUNK)])
                else:
                    pltpu.async_copy(rows_v, outb.at[wtail], sem).wait()

    return sc_gather


_SC_GATHER_CACHE = []


def kernel(layer_idx, text_states, text_mask, image_states, image_mask,
           cross_attn, previous_keep_mask):
    token_scores, gidx, new_mask = _tc_scores(cross_attn, text_mask, image_mask)
    if not _SC_GATHER_CACHE:
        _SC_GATHER_CACHE.append(_make_sc_gather())
    new_img_states = _SC_GATHER_CACHE[0](image_states, gidx)
    new_img_mask = new_mask.reshape(B, KP1)
    return (new_img_states, new_img_mask, previous_keep_mask, token_scores)


# speedup vs baseline: 2.0176x; 1.0262x over previous
import functools

import jax
import jax.numpy as jnp
from jax import lax
from jax.experimental import pallas as pl
from jax.experimental.pallas import tpu as pltpu
from jax.experimental.pallas import tpu_sc as plsc

B, T, L, H, D = 64, 64, 577, 12, 768
LNC = L - 1
K = LNC // 2
KP1 = K + 1
NIDX = 512
CHUNK = 128
NW = 32


def _score_kernel(ca_ref, tm_ref, im_ref, ts_ref, idx_ref, mask_ref):
    b = pl.program_id(0)
    X = ca_ref[0]

    acc = X[:, 0:8, :]
    for g in range(1, 8):
        acc = acc + X[:, 8 * g:8 * g + 8, :]
    t1 = acc[:, 4:8, :] + acc[:, 0:4, :]
    t2 = t1[:, 2:4, :] + t1[:, 0:2, :]
    S = t2[:, 1, :] + t2[:, 0, :]

    t_len = jnp.sum(tm_ref[...])
    Sd = S / t_len

    ts_ref[0] = Sd.T

    A = Sd[0:4, :] + Sd[8:12, :]
    a = Sd[4:8, :] + A
    bb = a[2:4, :] + a[0:2, :]
    c = bb[1:2, :] + bb[0:1, :]
    scores = c / jnp.float32(12.0)
    s_nc = scores[:, 1:]

    col = s_nc.T

    krow = jnp.broadcast_to(
        lax.bitcast_convert_type(s_nc, jnp.int32), (LNC, LNC))
    kcol = jnp.broadcast_to(
        lax.bitcast_convert_type(col, jnp.int32), (LNC, LNC))
    ii = lax.broadcasted_iota(jnp.int32, (LNC, LNC), 0)
    jj = lax.broadcasted_iota(jnp.int32, (LNC, LNC), 1)
    tri = jnp.where(jj < ii, 1, 0)
    G = (krow - kcol + tri) > 0
    ones = jnp.ones((LNC, 1), jnp.bfloat16)
    rank = lax.dot_general(G.astype(jnp.bfloat16), ones,
                           (((1,), (0,)), ((), ())),
                           preferred_element_type=jnp.float32)

    rr_l = lax.broadcasted_iota(jnp.int32, (LNC, NIDX), 1)
    rr = jnp.where(rr_l < KP1, rr_l, rr_l - KP1).astype(jnp.float32)
    P2 = (jnp.broadcast_to(rank + 1.0, (LNC, NIDX)) == rr).astype(jnp.bfloat16)

    idx_i = lax.broadcasted_iota(jnp.int32, (1, LNC), 1)
    idx_hi = (idx_i // 64).astype(jnp.bfloat16)
    idx_lo = (idx_i % 64).astype(jnp.bfloat16)
    mask_row = im_ref[0][:, 1:].astype(jnp.bfloat16)
    lhs = jnp.concatenate([idx_hi, idx_lo, mask_row], axis=0)
    packed = lax.dot_general(lhs, P2, (((1,), (0,)), ((), ())),
                             preferred_element_type=jnp.float32)
    top_shift = packed[0:1, :] * 64.0 + packed[1:2, :]
    gmask = packed[2:3, :]

    lane = lax.broadcasted_iota(jnp.int32, (1, NIDX), 1)
    lane_t = jnp.where(lane < KP1, lane, lane - KP1)
    gidx = jnp.where(lane_t == 0, 0.0, 1.0 + top_shift).astype(jnp.int32)
    idx_ref[...] = gidx[0]

    cls_mask = im_ref[0, 0, 0]
    mrow = jnp.where(lane == 0, cls_mask, gmask)
    mask_ref[0] = mrow[:, :KP1]


def _tc_scores(cross_attn, text_mask, image_mask):
    return pl.pallas_call(
        _score_kernel,
        grid=(B,),
        in_specs=[
            pl.BlockSpec((1, H, T, L), lambda b: (b, 0, 0, 0)),
            pl.BlockSpec((1, 1, T), lambda b: (b, 0, 0)),
            pl.BlockSpec((1, 1, L), lambda b: (b, 0, 0)),
        ],
        out_specs=[
            pl.BlockSpec((1, L, H), lambda b: (b, 0, 0)),
            pl.BlockSpec((NIDX,), lambda b: (b,)),
            pl.BlockSpec((1, 1, KP1), lambda b: (b, 0, 0)),
        ],
        out_shape=[
            jax.ShapeDtypeStruct((B, L, H), jnp.float32),
            jax.ShapeDtypeStruct((B * NIDX,), jnp.int32),
            jax.ShapeDtypeStruct((B, 1, KP1), jnp.float32),
        ],
    )(cross_attn, text_mask.reshape(B, 1, T), image_mask.reshape(B, 1, L))


GCH = 64
NCH = 5
NTAIL = KP1 - 4 * GCH


def _make_sc_gather():
    mesh = plsc.VectorSubcoreMesh(core_axis_name="c", subcore_axis_name="s")

    @functools.partial(
        pl.kernel,
        mesh=mesh,
        out_type=jax.ShapeDtypeStruct((B, KP1, D), jnp.float32),
        scratch_types=[
            pltpu.VMEM((3 * CHUNK,), jnp.int32),
            pltpu.VMEM((GCH, D), jnp.float32),
            pltpu.VMEM((GCH, D), jnp.float32),
            pltpu.VMEM((GCH,), jnp.int32),
            pltpu.SemaphoreType.DMA,
            pltpu.SemaphoreType.DMA,
        ],
    )
    def sc_gather(img_hbm, idx_hbm, out_hbm, idx_v, rows_a, rows_b, wtail,
                  sem_a, sem_b):
        w = lax.axis_index("s") * 2 + lax.axis_index("c")

        i16 = lax.broadcasted_iota(jnp.int32, (16,), 0)
        for k in range(GCH // 16):
            v = i16 + 16 * k
            wtail[pl.ds(16 * k, 16)] = jnp.where(
                v < NTAIL, v + 4 * GCH, v - NTAIL)

        bufs = (rows_a, rows_b)
        sems = (sem_a, sem_b)
        for bo in range(2):
            b = w * 2 + bo
            imgb = img_hbm.at[b]
            outb = out_hbm.at[b]
            for k in range(3):
                pltpu.sync_copy(
                    idx_hbm.at[pl.ds(b * NIDX + k * CHUNK, CHUNK)],
                    idx_v.at[pl.ds(k * CHUNK, CHUNK)])

            copies = []
            for ch in range(NCH):
                copies.append(pltpu.make_async_copy(
                    imgb.at[idx_v.at[pl.ds(ch * GCH, GCH)]],
                    bufs[ch % 2], sems[ch % 2]))
            copies[0].start()
            for ch in range(NCH):
                if ch + 1 < NCH:
                    copies[ch + 1].start()
                copies[ch].wait()
                if ch < NCH - 1:
                    pltpu.sync_copy(bufs[ch % 2],
                                    outb.at[pl.ds(ch * GCH, GCH)])
                else:
                    pltpu.async_copy(bufs[ch % 2], outb.at[wtail],
                                     sems[ch % 2]).wait()

    return sc_gather


_SC_GATHER_CACHE = []


def kernel(layer_idx, text_states, text_mask, image_states, image_mask,
           cross_attn, previous_keep_mask):
    token_scores, gidx, new_mask = _tc_scores(cross_attn, text_mask, image_mask)
    if not _SC_GATHER_CACHE:
        _SC_GATHER_CACHE.append(_make_sc_gather())
    new_img_states = _SC_GATHER_CACHE[0](image_states, gidx)
    new_img_mask = new_mask.reshape(B, KP1)
    return (new_img_states, new_img_mask, previous_keep_mask, token_scores)


# speedup vs baseline: 2.0199x; 1.0012x over previous
import functools

import jax
import jax.numpy as jnp
from jax import lax
from jax.experimental import pallas as pl
from jax.experimental.pallas import tpu as pltpu
from jax.experimental.pallas import tpu_sc as plsc

B, T, L, H, D = 64, 64, 577, 12, 768
LNC = L - 1
K = LNC // 2
KP1 = K + 1
NIDX = 512
CHUNK = 128
NW = 32


def _score_kernel(ca_ref, tm_ref, im_ref, ts_ref, idx_ref, mask_ref):
    b = pl.program_id(0)
    X = ca_ref[0]

    acc = X[:, 0:8, :]
    for g in range(1, 8):
        acc = acc + X[:, 8 * g:8 * g + 8, :]
    t1 = acc[:, 4:8, :] + acc[:, 0:4, :]
    t2 = t1[:, 2:4, :] + t1[:, 0:2, :]
    S = t2[:, 1, :] + t2[:, 0, :]

    t_len = jnp.sum(tm_ref[...])
    Sd = S / t_len

    ts_ref[0] = Sd.T

    A = Sd[0:4, :] + Sd[8:12, :]
    a = Sd[4:8, :] + A
    bb = a[2:4, :] + a[0:2, :]
    c = bb[1:2, :] + bb[0:1, :]
    scores = c / jnp.float32(12.0)
    s_nc = scores[:, 1:]

    col = s_nc.T

    krow = jnp.broadcast_to(
        lax.bitcast_convert_type(s_nc, jnp.int32), (LNC, LNC))
    kcol = jnp.broadcast_to(
        lax.bitcast_convert_type(col, jnp.int32), (LNC, LNC))
    ii = lax.broadcasted_iota(jnp.int32, (LNC, LNC), 0)
    jj = lax.broadcasted_iota(jnp.int32, (LNC, LNC), 1)
    tri = jnp.where(jj < ii, 1, 0)
    G = (krow - kcol + tri) > 0
    ones = jnp.ones((LNC, 1), jnp.int8)
    rank = lax.dot_general(G.astype(jnp.int8), ones,
                           (((1,), (0,)), ((), ())),
                           preferred_element_type=jnp.int32)

    NP = 384
    rr_l = lax.broadcasted_iota(jnp.int32, (LNC, NP), 1)
    rr = jnp.where(rr_l < KP1, rr_l, rr_l - KP1)
    P2 = (jnp.broadcast_to(rank + 1, (LNC, NP)) == rr).astype(jnp.int8)

    idx_i = lax.broadcasted_iota(jnp.int32, (1, LNC), 1)
    idx_hi = (idx_i // 64).astype(jnp.int8)
    idx_lo = (idx_i % 64).astype(jnp.int8)
    mask_row = im_ref[0][:, 1:].astype(jnp.int8)
    lhs = jnp.concatenate([idx_hi, idx_lo, mask_row], axis=0)
    packed = lax.dot_general(lhs, P2, (((1,), (0,)), ((), ())),
                             preferred_element_type=jnp.int32)
    top_shift = packed[0:1, :] * 64 + packed[1:2, :]
    gmask = packed[2:3, :].astype(jnp.float32)

    lane = lax.broadcasted_iota(jnp.int32, (1, NP), 1)
    lane_t = jnp.where(lane < KP1, lane, lane - KP1)
    gidx = jnp.where(lane_t == 0, 0, 1 + top_shift)
    gidx = jnp.concatenate(
        [gidx, jnp.zeros((1, NIDX - NP), jnp.int32)], axis=1)
    idx_ref[...] = gidx[0]

    cls_mask = im_ref[0, 0, 0]
    mrow = jnp.where(lane == 0, cls_mask, gmask)
    mask_ref[0] = mrow[:, :KP1]


def _tc_scores(cross_attn, text_mask, image_mask):
    return pl.pallas_call(
        _score_kernel,
        grid=(B,),
        in_specs=[
            pl.BlockSpec((1, H, T, L), lambda b: (b, 0, 0, 0)),
            pl.BlockSpec((1, 1, T), lambda b: (b, 0, 0)),
            pl.BlockSpec((1, 1, L), lambda b: (b, 0, 0)),
        ],
        out_specs=[
            pl.BlockSpec((1, L, H), lambda b: (b, 0, 0)),
            pl.BlockSpec((NIDX,), lambda b: (b,)),
            pl.BlockSpec((1, 1, KP1), lambda b: (b, 0, 0)),
        ],
        out_shape=[
            jax.ShapeDtypeStruct((B, L, H), jnp.float32),
            jax.ShapeDtypeStruct((B * NIDX,), jnp.int32),
            jax.ShapeDtypeStruct((B, 1, KP1), jnp.float32),
        ],
    )(cross_attn, text_mask.reshape(B, 1, T), image_mask.reshape(B, 1, L))


GCH = 64
NCH = 5
NTAIL = KP1 - 4 * GCH


def _make_sc_gather():
    mesh = plsc.VectorSubcoreMesh(core_axis_name="c", subcore_axis_name="s")

    @functools.partial(
        pl.kernel,
        mesh=mesh,
        out_type=jax.ShapeDtypeStruct((B, KP1, D), jnp.float32),
        scratch_types=[
            pltpu.VMEM((3 * CHUNK,), jnp.int32),
            pltpu.VMEM((GCH, D), jnp.float32),
            pltpu.VMEM((GCH, D), jnp.float32),
            pltpu.VMEM((GCH,), jnp.int32),
            pltpu.SemaphoreType.DMA,
            pltpu.SemaphoreType.DMA,
        ],
    )
    def sc_gather(img_hbm, idx_hbm, out_hbm, idx_v, rows_a, rows_b, wtail,
                  sem_a, sem_b):
        w = lax.axis_index("s") * 2 + lax.axis_index("c")

        i16 = lax.broadcasted_iota(jnp.int32, (16,), 0)
        for k in range(GCH // 16):
            v = i16 + 16 * k
            wtail[pl.ds(16 * k, 16)] = jnp.where(
                v < NTAIL, v + 4 * GCH, v - NTAIL)

        bufs = (rows_a, rows_b)
        sems = (sem_a, sem_b)
        for bo in range(2):
            b = w * 2 + bo
            imgb = img_hbm.at[b]
            outb = out_hbm.at[b]
            for k in range(3):
                pltpu.sync_copy(
                    idx_hbm.at[pl.ds(b * NIDX + k * CHUNK, CHUNK)],
                    idx_v.at[pl.ds(k * CHUNK, CHUNK)])

            copies = []
            for ch in range(NCH):
                copies.append(pltpu.make_async_copy(
                    imgb.at[idx_v.at[pl.ds(ch * GCH, GCH)]],
                    bufs[ch % 2], sems[ch % 2]))
            copies[0].start()
            for ch in range(NCH):
                if ch + 1 < NCH:
                    copies[ch + 1].start()
                copies[ch].wait()
                if ch < NCH - 1:
                    pltpu.sync_copy(bufs[ch % 2],
                                    outb.at[pl.ds(ch * GCH, GCH)])
                else:
                    pltpu.async_copy(bufs[ch % 2], outb.at[wtail],
                                     sems[ch % 2]).wait()

    return sc_gather


_SC_GATHER_CACHE = []


def kernel(layer_idx, text_states, text_mask, image_states, image_mask,
           cross_attn, previous_keep_mask):
    token_scores, gidx, new_mask = _tc_scores(cross_attn, text_mask, image_mask)
    if not _SC_GATHER_CACHE:
        _SC_GATHER_CACHE.append(_make_sc_gather())
    new_img_states = _SC_GATHER_CACHE[0](image_states, gidx)
    new_img_mask = new_mask.reshape(B, KP1)
    return (new_img_states, new_img_mask, previous_keep_mask, token_scores)
